# baseline (speedup 1.0000x reference)
import math

import jax
import jax.numpy as jnp
import numpy as np
from jax import lax
from jax.experimental import pallas as pl

_B = 16384
_DG = 16
_NBIN = 2
_CATK = [2, 2, 8, 8]
_T = 10
_H = 512
_DIN = 37
_DOUT = 36
_DXIN = 34

_RBLK = 256
_GRID = _B // _RBLK

_M32 = 0xFFFFFFFF


def _cosine_sched(timesteps, s=0.008):
    xs = np.linspace(0, timesteps, timesteps + 1)
    ab = np.cos(((xs / timesteps) + s) / (1 + s) * np.pi * 0.5) ** 2
    ab = ab / ab[0]
    betas = np.zeros(timesteps + 1)
    betas[1:] = 1.0 - ab[1:] / ab[:-1]
    betas = np.clip(betas, 1e-4, 0.999)
    alphas = 1.0 - betas
    return ab.astype(np.float32), alphas.astype(np.float32), betas.astype(np.float32)


_AB, _ALPHAS, _BETAS = _cosine_sched(_T)


def _tf2x32_host(k, x):
    k0, k1 = k
    x0, x1 = x
    ks = [k0, k1, (k0 ^ k1 ^ 0x1BD11BDA) & _M32]
    rot = [[13, 15, 26, 6], [17, 29, 16, 24]]
    x0 = (x0 + ks[0]) & _M32
    x1 = (x1 + ks[1]) & _M32

    def rounds(x0, x1, rs):
        for r in rs:
            x0 = (x0 + x1) & _M32
            x1 = ((x1 << r) | (x1 >> (32 - r))) & _M32
            x1 ^= x0
        return x0, x1

    for i in range(5):
        x0, x1 = rounds(x0, x1, rot[i % 2])
        x0 = (x0 + ks[(i + 1) % 3]) & _M32
        x1 = (x1 + ks[(i + 2) % 3] + i + 1) & _M32
    return x0, x1


_KEYS = [_tf2x32_host((0, 42), (0, c)) for c in range(5 * (_T + _T - 1))]

_LO = float(np.nextafter(np.float32(-1.0), np.float32(0.0)))
_SCALE_N = float(np.float32(1.0) - np.float32(_LO))
_TINY = float(np.finfo(np.float32).tiny)
_SQRT2 = float(np.float32(np.sqrt(2.0)))


def _tf2x32_vec(key, x1):
    k0 = jnp.uint32(key[0])
    k1 = jnp.uint32(key[1])
    k2 = jnp.uint32((key[0] ^ key[1] ^ 0x1BD11BDA) & _M32)
    ks = [k0, k1, k2]
    rot = [[13, 15, 26, 6], [17, 29, 16, 24]]
    x0 = jnp.full_like(x1, k0)
    x1 = x1 + k1

    def rounds(x0, x1, rs):
        for r in rs:
            x0 = x0 + x1
            x1 = (x1 << jnp.uint32(r)) | (x1 >> jnp.uint32(32 - r))
            x1 = x1 ^ x0
        return x0, x1

    for i in range(5):
        x0, x1 = rounds(x0, x1, rot[i % 2])
        x0 = x0 + ks[(i + 1) % 3]
        x1 = x1 + ks[(i + 2) % 3] + jnp.uint32(i + 1)
    return x0, x1


def _unit_floats(key, r0, rows, cols):
    row = lax.broadcasted_iota(jnp.uint32, (rows, cols), 0)
    col = lax.broadcasted_iota(jnp.uint32, (rows, cols), 1)
    e = (row + r0) * jnp.uint32(cols) + col
    y0, y1 = _tf2x32_vec(key, e)
    bits = y0 ^ y1
    fb = (bits >> jnp.uint32(9)) | jnp.uint32(0x3F800000)
    return lax.bitcast_convert_type(fb, jnp.float32) - jnp.float32(1.0)


def _normal_draw(key, r0, rows, cols):
    f = _unit_floats(key, r0, rows, cols)
    u = jnp.maximum(jnp.float32(_LO), f * jnp.float32(_SCALE_N) + jnp.float32(_LO))
    return jnp.float32(_SQRT2) * lax.erf_inv(u)


def _gumbel_draw(key, r0, rows, cols):
    f = _unit_floats(key, r0, rows, cols)
    u = jnp.maximum(jnp.float32(_TINY), f * jnp.float32(1.0) + jnp.float32(_TINY))
    return -jnp.log(-jnp.log(u))


def _layernorm(h, g, b):
    mu = jnp.mean(h, axis=-1, keepdims=True)
    var = jnp.var(h, axis=-1, keepdims=True)
    return (h - mu) / jnp.sqrt(var + 1e-5) * g + b


def _one_hot(idx, k):
    ioc = lax.broadcasted_iota(jnp.int32, (idx.shape[0], k), 1)
    return (ioc == idx[:, None]).astype(jnp.float32)


def _body(x_ref, W1_ref, b1_ref, g1_ref, be1_ref, W2_ref, b2_ref, g2_ref,
          be2_ref, W5_ref, b5_ref, out_ref):
    pid = pl.program_id(0)
    r0 = (pid * _RBLK).astype(jnp.uint32)
    xb = x_ref[...]

    xg = xb[:, :_DG]
    cats = []
    for j in range(_NBIN):
        idx = (xb[:, _DG + j] == 1.0).astype(jnp.int32)
        cats.append(_one_hot(idx, 2))
    off = _DG + _NBIN
    for k in [8, 8]:
        cats.append(xb[:, off:off + k])
        off += k

    eps = 1e-8
    kc = 0
    for t in range(1, _T + 1):
        beta = _BETAS[t]
        noise = _normal_draw(_KEYS[kc], r0, _RBLK, _DG)
        kc += 1
        xg = jnp.sqrt(1.0 - beta) * xg + jnp.sqrt(beta) * noise
        for i, k in enumerate(_CATK):
            probs = (1.0 - beta) * cats[i] + beta / k
            probs = probs / (probs.sum(axis=1, keepdims=True) + eps)
            gmb = _gumbel_draw(_KEYS[kc], r0, _RBLK, k)
            kc += 1
            idx = jnp.argmax(jnp.log(probs + 1e-12) + gmb, axis=1)
            cats[i] = _one_hot(idx, k)

    for t in range(_T - 1, 0, -1):
        z = jnp.concatenate([xg] + cats, axis=1)
        tcol = jnp.full((_RBLK, 1), math.sin(t * 1000.0), dtype=jnp.float32)
        h = jnp.concatenate([z, tcol], axis=1)
        h = jnp.dot(h, W1_ref[t], preferred_element_type=jnp.float32) + b1_ref[t]
        h = jax.nn.gelu(_layernorm(h, g1_ref[t], be1_ref[t]))
        h = jnp.dot(h, W2_ref[t], preferred_element_type=jnp.float32) + b2_ref[t]
        h = jax.nn.gelu(_layernorm(h, g2_ref[t], be2_ref[t]))
        out = jnp.dot(h, W5_ref[t], preferred_element_type=jnp.float32) + b5_ref[t]

        noise_hat = jnp.clip(out[:, :_DG], -5.0, 5.0)
        alpha = _ALPHAS[t]
        beta = _BETAS[t]
        ab_t = _AB[t]
        ab_tm1 = _AB[t - 1]
        mu = (xg - beta / jnp.sqrt(1.0 - ab_t) * noise_hat) / jnp.sqrt(alpha)
        sigma = jnp.maximum(jnp.sqrt(beta * (1.0 - ab_tm1) / (1.0 - ab_t)), eps)
        noise = _normal_draw(_KEYS[kc], r0, _RBLK, _DG)
        kc += 1
        xg = mu + sigma * noise

        coff = _DG
        for i, k in enumerate(_CATK):
            cat_hat = jax.nn.softmax(out[:, coff:coff + k], axis=1)
            coff += k
            pi = (alpha * cats[i] + (1.0 - alpha) / k) * (ab_tm1 * cat_hat + (1.0 - ab_tm1) / k)
            pi = pi / (pi.sum(axis=1, keepdims=True) + eps)
            pi = jnp.maximum(pi, 1e-6)
            gmb = _gumbel_draw(_KEYS[kc], r0, _RBLK, k)
            kc += 1
            idx = jnp.argmax(jnp.log(pi) + gmb, axis=1)
            cats[i] = _one_hot(idx, k)

    out_ref[...] = jnp.concatenate([xg] + cats, axis=1)


def kernel(x, W1, b1, g1, be1, W2, b2, g2, be2, W5, b5):
    full = lambda shape: pl.BlockSpec(shape, lambda i: (0,) * len(shape))
    return pl.pallas_call(
        _body,
        grid=(_GRID,),
        in_specs=[
            pl.BlockSpec((_RBLK, _DXIN), lambda i: (i, 0)),
            full((_T, _DIN, _H)), full((_T, _H)), full((_T, _H)), full((_T, _H)),
            full((_T, _H, _H)), full((_T, _H)), full((_T, _H)), full((_T, _H)),
            full((_T, _H, _DOUT + 1)), full((_T, _DOUT + 1)),
        ],
        out_specs=pl.BlockSpec((_RBLK, _DOUT), lambda i: (i, 0)),
        out_shape=jax.ShapeDtypeStruct((_B, _DOUT), jnp.float32),
    )(x, W1, b1, g1, be1, W2, b2, g2, be2, W5, b5)


# speedup vs baseline: 5.1616x; 5.1616x over previous
import math

import jax
import jax.numpy as jnp
import numpy as np
from jax import lax
from jax.experimental import pallas as pl

_B = 16384
_DG = 16
_NBIN = 2
_CATK = [2, 2, 8, 8]
_T = 10
_H = 512
_DIN = 37
_DOUT = 36
_DXIN = 34

_RBLK = 256
_GRID = _B // _RBLK

_M32 = 0xFFFFFFFF


def _cosine_sched(timesteps, s=0.008):
    xs = np.linspace(0, timesteps, timesteps + 1)
    ab = np.cos(((xs / timesteps) + s) / (1 + s) * np.pi * 0.5) ** 2
    ab = ab / ab[0]
    betas = np.zeros(timesteps + 1)
    betas[1:] = 1.0 - ab[1:] / ab[:-1]
    betas = np.clip(betas, 1e-4, 0.999)
    alphas = 1.0 - betas
    return ab.astype(np.float32), alphas.astype(np.float32), betas.astype(np.float32)


_AB, _ALPHAS, _BETAS = _cosine_sched(_T)


def _tf2x32_host(k, x):
    k0, k1 = k
    x0, x1 = x
    ks = [k0, k1, (k0 ^ k1 ^ 0x1BD11BDA) & _M32]
    rot = [[13, 15, 26, 6], [17, 29, 16, 24]]
    x0 = (x0 + ks[0]) & _M32
    x1 = (x1 + ks[1]) & _M32

    def rounds(x0, x1, rs):
        for r in rs:
            x0 = (x0 + x1) & _M32
            x1 = ((x1 << r) | (x1 >> (32 - r))) & _M32
            x1 ^= x0
        return x0, x1

    for i in range(5):
        x0, x1 = rounds(x0, x1, rot[i % 2])
        x0 = (x0 + ks[(i + 1) % 3]) & _M32
        x1 = (x1 + ks[(i + 2) % 3] + i + 1) & _M32
    return x0, x1


_KEYS = [_tf2x32_host((0, 42), (0, c)) for c in range(5 * (_T + _T - 1))]

_LO = float(np.nextafter(np.float32(-1.0), np.float32(0.0)))
_SCALE_N = float(np.float32(1.0) - np.float32(_LO))
_TINY = float(np.finfo(np.float32).tiny)
_SQRT2 = float(np.float32(np.sqrt(2.0)))

_SLAB_W = 384
_GOFF = [0, 2, 4, 12]

_NK0 = np.zeros((1, _SLAB_W), np.uint32)
_NK1 = np.zeros((1, _SLAB_W), np.uint32)
_NSTRIDE = np.zeros((1, _SLAB_W), np.uint32)
_NOFF = np.zeros((1, _SLAB_W), np.uint32)
_GK0 = np.zeros((1, _SLAB_W), np.uint32)
_GK1 = np.zeros((1, _SLAB_W), np.uint32)
_GSTRIDE = np.zeros((1, _SLAB_W), np.uint32)
_GOFFC = np.zeros((1, _SLAB_W), np.uint32)
for _n in range(19):
    _k = _KEYS[5 * _n]
    for _c in range(_DG):
        _j = 16 * _n + _c
        _NK0[0, _j], _NK1[0, _j] = _k
        _NSTRIDE[0, _j] = _DG
        _NOFF[0, _j] = _c
for _s in range(19):
    for _i, _kk in enumerate(_CATK):
        _k = _KEYS[5 * _s + 1 + _i]
        for _c in range(_kk):
            _j = 20 * _s + _GOFF[_i] + _c
            _GK0[0, _j], _GK1[0, _j] = _k
            _GSTRIDE[0, _j] = _kk
            _GOFFC[0, _j] = _c

_RNGC = np.concatenate([_NK0, _NK1, _NSTRIDE, _NOFF], axis=0)
_RNGT = np.concatenate([_GK0, _GK1, _GSTRIDE, _GOFFC], axis=0).T.copy()


def _tf2x32_lanes(k0, k1, x1):
    k2 = k0 ^ k1 ^ np.uint32(0x1BD11BDA)
    ks = [k0, k1, k2]
    rot = [[13, 15, 26, 6], [17, 29, 16, 24]]
    x0 = jnp.broadcast_to(k0, x1.shape)
    x1 = x1 + k1

    def rounds(x0, x1, rs):
        for r in rs:
            x0 = x0 + x1
            x1 = (x1 << jnp.uint32(r)) | (x1 >> jnp.uint32(32 - r))
            x1 = x1 ^ x0
        return x0, x1

    for i in range(5):
        x0, x1 = rounds(x0, x1, rot[i % 2])
        x0 = x0 + ks[(i + 1) % 3]
        x1 = x1 + ks[(i + 2) % 3] + jnp.uint32(i + 1)
    return x0, x1


def _bits_to_unit(bits):
    fb = (bits >> jnp.uint32(9)) | jnp.uint32(0x3F800000)
    return lax.bitcast_convert_type(fb, jnp.float32) - jnp.float32(1.0)


def _normal_slab(rngc, r0, rows):
    row = lax.broadcasted_iota(jnp.uint32, (rows, _SLAB_W), 0) + r0
    e = row * rngc[2:3, :] + rngc[3:4, :]
    y0, y1 = _tf2x32_lanes(rngc[0:1, :], rngc[1:2, :], e)
    f = _bits_to_unit(y0 ^ y1)
    u = jnp.maximum(jnp.float32(_LO), f * jnp.float32(_SCALE_N) + jnp.float32(_LO))
    return jnp.float32(_SQRT2) * lax.erf_inv(u)


def _gumbel_slab_t(rngt, r0, rows):
    lane = lax.broadcasted_iota(jnp.uint32, (_SLAB_W, rows), 1) + r0
    e = lane * rngt[:, 2:3] + rngt[:, 3:4]
    y0, y1 = _tf2x32_lanes(rngt[:, 0:1], rngt[:, 1:2], e)
    f = _bits_to_unit(y0 ^ y1)
    u = jnp.maximum(jnp.float32(_TINY), f * jnp.float32(1.0) + jnp.float32(_TINY))
    return -jnp.log(-jnp.log(u))


def _layernorm(h, g, b):
    mu = jnp.mean(h, axis=-1, keepdims=True)
    var = jnp.var(h, axis=-1, keepdims=True)
    return (h - mu) / jnp.sqrt(var + 1e-5) * g + b


def _one_hot_t(idx, k):
    ior = lax.broadcasted_iota(jnp.int32, (k, idx.shape[0]), 0)
    return (ior == idx[None, :]).astype(jnp.float32)


def _cat_sample_t(pi_logged, gmb, k):
    idx = jnp.argmax(pi_logged + gmb, axis=0)
    return _one_hot_t(idx, k)


def _body(x_ref, W1_ref, b1_ref, g1_ref, be1_ref, W2_ref, b2_ref, g2_ref,
          be2_ref, W5_ref, b5_ref, rngc_ref, rngt_ref, out_ref):
    pid = pl.program_id(0)
    r0 = (pid * _RBLK).astype(jnp.uint32)
    xb = x_ref[...]

    xg = xb[:, :_DG]
    xbT = xb[:, _DG:_DG + 18].T
    cats = []
    for j in range(_NBIN):
        b1r = (xbT[j:j + 1, :] == 1.0).astype(jnp.float32)
        cats.append(jnp.concatenate([1.0 - b1r, b1r], axis=0))
    cats.append(xbT[2:10, :])
    cats.append(xbT[10:18, :])

    normals = _normal_slab(rngc_ref[...], r0, _RBLK)
    gumbelsT = _gumbel_slab_t(rngt_ref[...], r0, _RBLK)

    eps = 1e-8
    for t in range(1, _T + 1):
        beta = _BETAS[t]
        s = t - 1
        noise = normals[:, 16 * s:16 * s + _DG]
        xg = jnp.sqrt(1.0 - beta) * xg + jnp.sqrt(beta) * noise
        for i, k in enumerate(_CATK):
            probs = (1.0 - beta) * cats[i] + beta / k
            probs = probs / (probs.sum(axis=0, keepdims=True) + eps)
            g0 = 20 * s + _GOFF[i]
            cats[i] = _cat_sample_t(jnp.log(probs + 1e-12),
                                    gumbelsT[g0:g0 + k, :], k)

    for t in range(_T - 1, 0, -1):
        catR = jnp.concatenate(cats, axis=0).T
        tcol = jnp.full((_RBLK, 1), math.sin(t * 1000.0), dtype=jnp.float32)
        h = jnp.concatenate([xg, catR, tcol], axis=1)
        h = jnp.dot(h, W1_ref[t], preferred_element_type=jnp.float32) + b1_ref[t]
        h = jax.nn.gelu(_layernorm(h, g1_ref[t], be1_ref[t]))
        h = jnp.dot(h, W2_ref[t], preferred_element_type=jnp.float32) + b2_ref[t]
        h = jax.nn.gelu(_layernorm(h, g2_ref[t], be2_ref[t]))
        out = jnp.dot(h, W5_ref[t], preferred_element_type=jnp.float32) + b5_ref[t]

        noise_hat = jnp.clip(out[:, :_DG], -5.0, 5.0)
        alpha = _ALPHAS[t]
        beta = _BETAS[t]
        ab_t = _AB[t]
        ab_tm1 = _AB[t - 1]
        mu = (xg - beta / jnp.sqrt(1.0 - ab_t) * noise_hat) / jnp.sqrt(alpha)
        sigma = jnp.maximum(jnp.sqrt(beta * (1.0 - ab_tm1) / (1.0 - ab_t)), eps)
        s = _T + (_T - 1 - t)
        noise = normals[:, 16 * s:16 * s + _DG]
        xg = mu + sigma * noise

        outT = out[:, _DG:_DG + 20].T
        for i, k in enumerate(_CATK):
            o0 = _GOFF[i]
            seg = outT[o0:o0 + k, :]
            m = jnp.max(seg, axis=0, keepdims=True)
            unn = jnp.exp(seg - m)
            cat_hat = unn / unn.sum(axis=0, keepdims=True)
            pi = (alpha * cats[i] + (1.0 - alpha) / k) * (ab_tm1 * cat_hat + (1.0 - ab_tm1) / k)
            pi = pi / (pi.sum(axis=0, keepdims=True) + eps)
            pi = jnp.maximum(pi, 1e-6)
            g0 = 20 * s + _GOFF[i]
            cats[i] = _cat_sample_t(jnp.log(pi), gumbelsT[g0:g0 + k, :], k)

    out_ref[...] = jnp.concatenate([xg, jnp.concatenate(cats, axis=0).T], axis=1)


def kernel(x, W1, b1, g1, be1, W2, b2, g2, be2, W5, b5):
    full = lambda shape: pl.BlockSpec(shape, lambda i: (0,) * len(shape))
    return pl.pallas_call(
        _body,
        grid=(_GRID,),
        in_specs=[
            pl.BlockSpec((_RBLK, _DXIN), lambda i: (i, 0)),
            full((_T, _DIN, _H)), full((_T, _H)), full((_T, _H)), full((_T, _H)),
            full((_T, _H, _H)), full((_T, _H)), full((_T, _H)), full((_T, _H)),
            full((_T, _H, _DOUT + 1)), full((_T, _DOUT + 1)),
            full((4, _SLAB_W)), full((_SLAB_W, 4)),
        ],
        out_specs=pl.BlockSpec((_RBLK, _DOUT), lambda i: (i, 0)),
        out_shape=jax.ShapeDtypeStruct((_B, _DOUT), jnp.float32),
    )(x, W1, b1, g1, be1, W2, b2, g2, be2, W5, b5,
      jnp.asarray(_RNGC), jnp.asarray(_RNGT))


# speedup vs baseline: 6.5125x; 1.2617x over previous
import math

import jax
import jax.numpy as jnp
import numpy as np
from jax import lax
from jax.experimental import pallas as pl

_B = 16384
_DG = 16
_NBIN = 2
_CATK = [2, 2, 8, 8]
_T = 10
_H = 512
_DIN = 37
_DOUT = 36
_DXIN = 34

_RBLK = 512
_GRID = _B // _RBLK

_M32 = 0xFFFFFFFF


def _cosine_sched(timesteps, s=0.008):
    xs = np.linspace(0, timesteps, timesteps + 1)
    ab = np.cos(((xs / timesteps) + s) / (1 + s) * np.pi * 0.5) ** 2
    ab = ab / ab[0]
    betas = np.zeros(timesteps + 1)
    betas[1:] = 1.0 - ab[1:] / ab[:-1]
    betas = np.clip(betas, 1e-4, 0.999)
    alphas = 1.0 - betas
    return ab.astype(np.float32), alphas.astype(np.float32), betas.astype(np.float32)


_AB, _ALPHAS, _BETAS = _cosine_sched(_T)


def _tf2x32_host(k, x):
    k0, k1 = k
    x0, x1 = x
    ks = [k0, k1, (k0 ^ k1 ^ 0x1BD11BDA) & _M32]
    rot = [[13, 15, 26, 6], [17, 29, 16, 24]]
    x0 = (x0 + ks[0]) & _M32
    x1 = (x1 + ks[1]) & _M32

    def rounds(x0, x1, rs):
        for r in rs:
            x0 = (x0 + x1) & _M32
            x1 = ((x1 << r) | (x1 >> (32 - r))) & _M32
            x1 ^= x0
        return x0, x1

    for i in range(5):
        x0, x1 = rounds(x0, x1, rot[i % 2])
        x0 = (x0 + ks[(i + 1) % 3]) & _M32
        x1 = (x1 + ks[(i + 2) % 3] + i + 1) & _M32
    return x0, x1


_KEYS = [_tf2x32_host((0, 42), (0, c)) for c in range(5 * (_T + _T - 1))]

_LO = float(np.nextafter(np.float32(-1.0), np.float32(0.0)))
_SCALE_N = float(np.float32(1.0) - np.float32(_LO))
_TINY = float(np.finfo(np.float32).tiny)
_SQRT2 = float(np.float32(np.sqrt(2.0)))

_SLAB_W = 384
_GOFF = [0, 2, 4, 12]

_NK0 = np.zeros((1, _SLAB_W), np.uint32)
_NK1 = np.zeros((1, _SLAB_W), np.uint32)
_NSTRIDE = np.zeros((1, _SLAB_W), np.uint32)
_NOFF = np.zeros((1, _SLAB_W), np.uint32)
_GK0 = np.zeros((1, _SLAB_W), np.uint32)
_GK1 = np.zeros((1, _SLAB_W), np.uint32)
_GSTRIDE = np.zeros((1, _SLAB_W), np.uint32)
_GOFFC = np.zeros((1, _SLAB_W), np.uint32)
for _n in range(19):
    _k = _KEYS[5 * _n]
    for _c in range(_DG):
        _j = 16 * _n + _c
        _NK0[0, _j], _NK1[0, _j] = _k
        _NSTRIDE[0, _j] = _DG
        _NOFF[0, _j] = _c
for _s in range(19):
    for _i, _kk in enumerate(_CATK):
        _k = _KEYS[5 * _s + 1 + _i]
        for _c in range(_kk):
            _j = 20 * _s + _GOFF[_i] + _c
            _GK0[0, _j], _GK1[0, _j] = _k
            _GSTRIDE[0, _j] = _kk
            _GOFFC[0, _j] = _c

_RNGC = np.concatenate([_NK0, _NK1, _NSTRIDE, _NOFF], axis=0)
_RNGT = np.concatenate([_GK0, _GK1, _GSTRIDE, _GOFFC], axis=0).T.copy()


def _tf2x32_lanes(k0, k1, x1):
    k2 = k0 ^ k1 ^ np.uint32(0x1BD11BDA)
    ks = [k0, k1, k2]
    rot = [[13, 15, 26, 6], [17, 29, 16, 24]]
    x0 = jnp.broadcast_to(k0, x1.shape)
    x1 = x1 + k1

    def rounds(x0, x1, rs):
        for r in rs:
            x0 = x0 + x1
            x1 = (x1 << jnp.uint32(r)) | (x1 >> jnp.uint32(32 - r))
            x1 = x1 ^ x0
        return x0, x1

    for i in range(5):
        x0, x1 = rounds(x0, x1, rot[i % 2])
        x0 = x0 + ks[(i + 1) % 3]
        x1 = x1 + ks[(i + 2) % 3] + jnp.uint32(i + 1)
    return x0, x1


def _bits_to_unit(bits):
    fb = (bits >> jnp.uint32(9)) | jnp.uint32(0x3F800000)
    return lax.bitcast_convert_type(fb, jnp.float32) - jnp.float32(1.0)


def _normal_slab(rngc, r0, rows):
    row = lax.broadcasted_iota(jnp.uint32, (rows, _SLAB_W), 0) + r0
    e = row * rngc[2:3, :] + rngc[3:4, :]
    y0, y1 = _tf2x32_lanes(rngc[0:1, :], rngc[1:2, :], e)
    f = _bits_to_unit(y0 ^ y1)
    u = jnp.maximum(jnp.float32(_LO), f * jnp.float32(_SCALE_N) + jnp.float32(_LO))
    return jnp.float32(_SQRT2) * lax.erf_inv(u)


def _gumbel_slab_t(rngt, r0, rows):
    lane = lax.broadcasted_iota(jnp.uint32, (_SLAB_W, rows), 1) + r0
    e = lane * rngt[:, 2:3] + rngt[:, 3:4]
    y0, y1 = _tf2x32_lanes(rngt[:, 0:1], rngt[:, 1:2], e)
    f = _bits_to_unit(y0 ^ y1)
    u = jnp.maximum(jnp.float32(_TINY), f * jnp.float32(1.0) + jnp.float32(_TINY))
    return -jnp.log(-jnp.log(u))


def _layernorm(h, g, b):
    mu = jnp.mean(h, axis=-1, keepdims=True)
    var = jnp.var(h, axis=-1, keepdims=True)
    return (h - mu) / jnp.sqrt(var + 1e-5) * g + b


def _one_hot_t(idx, k):
    ior = lax.broadcasted_iota(jnp.int32, (k, idx.shape[0]), 0)
    return (ior == idx[None, :]).astype(jnp.float32)


def _cat_sample_t(pi_logged, gmb, k):
    idx = jnp.argmax(pi_logged + gmb, axis=0)
    return _one_hot_t(idx, k)


def _body(x_ref, W1_ref, b1_ref, g1_ref, be1_ref, W2_ref, b2_ref, g2_ref,
          be2_ref, W5_ref, b5_ref, rngc_ref, rngt_ref, out_ref):
    pid = pl.program_id(0)
    r0 = (pid * _RBLK).astype(jnp.uint32)
    xb = x_ref[...]

    xg = xb[:, :_DG]
    xbT = xb[:, _DG:_DG + 18].T
    cats = []
    for j in range(_NBIN):
        b1r = (xbT[j:j + 1, :] == 1.0).astype(jnp.float32)
        cats.append(jnp.concatenate([1.0 - b1r, b1r], axis=0))
    cats.append(xbT[2:10, :])
    cats.append(xbT[10:18, :])

    normals = _normal_slab(rngc_ref[...], r0, _RBLK)
    gumbelsT = _gumbel_slab_t(rngt_ref[...], r0, _RBLK)

    eps = 1e-8
    for t in range(1, _T + 1):
        beta = _BETAS[t]
        s = t - 1
        noise = normals[:, 16 * s:16 * s + _DG]
        xg = jnp.sqrt(1.0 - beta) * xg + jnp.sqrt(beta) * noise
        for i, k in enumerate(_CATK):
            probs = (1.0 - beta) * cats[i] + beta / k
            probs = probs / (probs.sum(axis=0, keepdims=True) + eps)
            g0 = 20 * s + _GOFF[i]
            cats[i] = _cat_sample_t(jnp.log(probs + 1e-12),
                                    gumbelsT[g0:g0 + k, :], k)

    for t in range(_T - 1, 0, -1):
        catR = jnp.concatenate(cats, axis=0).T
        tcol = jnp.full((_RBLK, 1), math.sin(t * 1000.0), dtype=jnp.float32)
        h = jnp.concatenate([xg, catR, tcol], axis=1)
        h = jnp.dot(h, W1_ref[t], preferred_element_type=jnp.float32) + b1_ref[t]
        h = jax.nn.gelu(_layernorm(h, g1_ref[t], be1_ref[t]))
        h = jnp.dot(h, W2_ref[t], preferred_element_type=jnp.float32) + b2_ref[t]
        h = jax.nn.gelu(_layernorm(h, g2_ref[t], be2_ref[t]))
        out = jnp.dot(h, W5_ref[t], preferred_element_type=jnp.float32) + b5_ref[t]

        noise_hat = jnp.clip(out[:, :_DG], -5.0, 5.0)
        alpha = _ALPHAS[t]
        beta = _BETAS[t]
        ab_t = _AB[t]
        ab_tm1 = _AB[t - 1]
        mu = (xg - beta / jnp.sqrt(1.0 - ab_t) * noise_hat) / jnp.sqrt(alpha)
        sigma = jnp.maximum(jnp.sqrt(beta * (1.0 - ab_tm1) / (1.0 - ab_t)), eps)
        s = _T + (_T - 1 - t)
        noise = normals[:, 16 * s:16 * s + _DG]
        xg = mu + sigma * noise

        outT = out[:, _DG:_DG + 20].T
        for i, k in enumerate(_CATK):
            o0 = _GOFF[i]
            seg = outT[o0:o0 + k, :]
            m = jnp.max(seg, axis=0, keepdims=True)
            unn = jnp.exp(seg - m)
            cat_hat = unn / unn.sum(axis=0, keepdims=True)
            pi = (alpha * cats[i] + (1.0 - alpha) / k) * (ab_tm1 * cat_hat + (1.0 - ab_tm1) / k)
            pi = pi / (pi.sum(axis=0, keepdims=True) + eps)
            pi = jnp.maximum(pi, 1e-6)
            g0 = 20 * s + _GOFF[i]
            cats[i] = _cat_sample_t(jnp.log(pi), gumbelsT[g0:g0 + k, :], k)

    out_ref[...] = jnp.concatenate([xg, jnp.concatenate(cats, axis=0).T], axis=1)


def kernel(x, W1, b1, g1, be1, W2, b2, g2, be2, W5, b5):
    full = lambda shape: pl.BlockSpec(shape, lambda i: (0,) * len(shape))
    return pl.pallas_call(
        _body,
        grid=(_GRID,),
        in_specs=[
            pl.BlockSpec((_RBLK, _DXIN), lambda i: (i, 0)),
            full((_T, _DIN, _H)), full((_T, _H)), full((_T, _H)), full((_T, _H)),
            full((_T, _H, _H)), full((_T, _H)), full((_T, _H)), full((_T, _H)),
            full((_T, _H, _DOUT + 1)), full((_T, _DOUT + 1)),
            full((4, _SLAB_W)), full((_SLAB_W, 4)),
        ],
        out_specs=pl.BlockSpec((_RBLK, _DOUT), lambda i: (i, 0)),
        out_shape=jax.ShapeDtypeStruct((_B, _DOUT), jnp.float32),
    )(x, W1, b1, g1, be1, W2, b2, g2, be2, W5, b5,
      jnp.asarray(_RNGC), jnp.asarray(_RNGT))


# speedup vs baseline: 7.2278x; 1.1098x over previous
import math

import jax
import jax.numpy as jnp
import numpy as np
from jax import lax
from jax.experimental import pallas as pl

_B = 16384
_DG = 16
_NBIN = 2
_CATK = [2, 2, 8, 8]
_T = 10
_H = 512
_DIN = 37
_DOUT = 36
_DXIN = 34

_RBLK = 512
_GRID = _B // _RBLK

_M32 = 0xFFFFFFFF


def _cosine_sched(timesteps, s=0.008):
    xs = np.linspace(0, timesteps, timesteps + 1)
    ab = np.cos(((xs / timesteps) + s) / (1 + s) * np.pi * 0.5) ** 2
    ab = ab / ab[0]
    betas = np.zeros(timesteps + 1)
    betas[1:] = 1.0 - ab[1:] / ab[:-1]
    betas = np.clip(betas, 1e-4, 0.999)
    alphas = 1.0 - betas
    return ab.astype(np.float32), alphas.astype(np.float32), betas.astype(np.float32)


_AB, _ALPHAS, _BETAS = _cosine_sched(_T)


def _tf2x32_host(k, x):
    k0, k1 = k
    x0, x1 = x
    ks = [k0, k1, (k0 ^ k1 ^ 0x1BD11BDA) & _M32]
    rot = [[13, 15, 26, 6], [17, 29, 16, 24]]
    x0 = (x0 + ks[0]) & _M32
    x1 = (x1 + ks[1]) & _M32

    def rounds(x0, x1, rs):
        for r in rs:
            x0 = (x0 + x1) & _M32
            x1 = ((x1 << r) | (x1 >> (32 - r))) & _M32
            x1 ^= x0
        return x0, x1

    for i in range(5):
        x0, x1 = rounds(x0, x1, rot[i % 2])
        x0 = (x0 + ks[(i + 1) % 3]) & _M32
        x1 = (x1 + ks[(i + 2) % 3] + i + 1) & _M32
    return x0, x1


_KEYS = [_tf2x32_host((0, 42), (0, c)) for c in range(5 * (_T + _T - 1))]

_LO = float(np.nextafter(np.float32(-1.0), np.float32(0.0)))
_SCALE_N = float(np.float32(1.0) - np.float32(_LO))
_TINY = float(np.finfo(np.float32).tiny)
_SQRT2 = float(np.float32(np.sqrt(2.0)))

_NW = 304
_GW = 384
_GOFF = [0, 2, 4, 12]

_RNGN = np.zeros((_NW, 4), np.uint32)
_RNGT = np.zeros((_GW, 4), np.uint32)
for _n in range(19):
    _k = _KEYS[5 * _n]
    for _c in range(_DG):
        _RNGN[16 * _n + _c] = (_k[0], _k[1], _DG, _c)
for _s in range(19):
    for _i, _kk in enumerate(_CATK):
        _k = _KEYS[5 * _s + 1 + _i]
        for _c in range(_kk):
            _RNGT[20 * _s + _GOFF[_i] + _c] = (_k[0], _k[1], _kk, _c)


def _tf2x32_lanes(k0, k1, x1):
    k2 = k0 ^ k1 ^ np.uint32(0x1BD11BDA)
    ks = [k0, k1, k2]
    rot = [[13, 15, 26, 6], [17, 29, 16, 24]]
    x0 = jnp.broadcast_to(k0, x1.shape)
    x1 = x1 + k1

    def rounds(x0, x1, rs):
        for r in rs:
            x0 = x0 + x1
            x1 = (x1 << jnp.uint32(r)) | (x1 >> jnp.uint32(32 - r))
            x1 = x1 ^ x0
        return x0, x1

    for i in range(5):
        x0, x1 = rounds(x0, x1, rot[i % 2])
        x0 = x0 + ks[(i + 1) % 3]
        x1 = x1 + ks[(i + 2) % 3] + jnp.uint32(i + 1)
    return x0, x1


def _bits_to_unit(bits):
    fb = (bits >> jnp.uint32(9)) | jnp.uint32(0x3F800000)
    return lax.bitcast_convert_type(fb, jnp.float32) - jnp.float32(1.0)


def _counter_bits_t(rngt, w, r0, rows):
    lane = lax.broadcasted_iota(jnp.uint32, (w, rows), 1) + r0
    e = lane * rngt[:, 2:3] + rngt[:, 3:4]
    y0, y1 = _tf2x32_lanes(rngt[:, 0:1], rngt[:, 1:2], e)
    return _bits_to_unit(y0 ^ y1)


def _normal_slab_t(rngn, r0, rows):
    f = _counter_bits_t(rngn, _NW, r0, rows)
    u = jnp.maximum(jnp.float32(_LO), f * jnp.float32(_SCALE_N) + jnp.float32(_LO))
    return jnp.float32(_SQRT2) * lax.erf_inv(u)


def _gumbel_slab_t(rngt, r0, rows):
    f = _counter_bits_t(rngt, _GW, r0, rows)
    u = jnp.maximum(jnp.float32(_TINY), f * jnp.float32(1.0) + jnp.float32(_TINY))
    return -jnp.log(-jnp.log(u))


def _layernorm(h, g, b):
    mu = jnp.mean(h, axis=-1, keepdims=True)
    var = jnp.var(h, axis=-1, keepdims=True)
    return (h - mu) / jnp.sqrt(var + 1e-5) * g + b


def _one_hot_t(idx, k):
    ior = lax.broadcasted_iota(jnp.int32, (k, idx.shape[0]), 0)
    return (ior == idx[None, :]).astype(jnp.float32)


def _cat_sample_t(pi_logged, gmb, k):
    idx = jnp.argmax(pi_logged + gmb, axis=0)
    return _one_hot_t(idx, k)


def _body(x_ref, W1_ref, b1_ref, g1_ref, be1_ref, W2_ref, b2_ref, g2_ref,
          be2_ref, W5_ref, b5_ref, rngn_ref, rngt_ref, out_ref):
    pid = pl.program_id(0)
    r0 = (pid * _RBLK).astype(jnp.uint32)
    xbT = x_ref[...].T

    xgT = xbT[:_DG, :]
    cats = []
    for j in range(_NBIN):
        b1r = (xbT[_DG + j:_DG + j + 1, :] == 1.0).astype(jnp.float32)
        cats.append(jnp.concatenate([1.0 - b1r, b1r], axis=0))
    cats.append(xbT[18:26, :])
    cats.append(xbT[26:34, :])

    normalsT = _normal_slab_t(rngn_ref[...], r0, _RBLK)
    gumbelsT = _gumbel_slab_t(rngt_ref[...], r0, _RBLK)

    eps = 1e-8
    for t in range(1, _T + 1):
        beta = _BETAS[t]
        s = t - 1
        noiseT = normalsT[16 * s:16 * s + _DG, :]
        xgT = jnp.sqrt(1.0 - beta) * xgT + jnp.sqrt(beta) * noiseT
        for i, k in enumerate(_CATK):
            probs = (1.0 - beta) * cats[i] + beta / k
            probs = probs / (probs.sum(axis=0, keepdims=True) + eps)
            g0 = 20 * s + _GOFF[i]
            cats[i] = _cat_sample_t(jnp.log(probs + 1e-12),
                                    gumbelsT[g0:g0 + k, :], k)

    for t in range(_T - 1, 0, -1):
        tcolT = jnp.full((1, _RBLK), math.sin(t * 1000.0), dtype=jnp.float32)
        h = jnp.concatenate([xgT] + cats + [tcolT], axis=0).T
        h = jnp.dot(h, W1_ref[t], preferred_element_type=jnp.float32) + b1_ref[t]
        h = _layernorm(h, g1_ref[t], be1_ref[t])
        h = jax.nn.gelu(h)
        h = jnp.dot(h, W2_ref[t], preferred_element_type=jnp.float32) + b2_ref[t]
        h = _layernorm(h, g2_ref[t], be2_ref[t])
        h = jax.nn.gelu(h)
        out = jnp.dot(h, W5_ref[t], preferred_element_type=jnp.float32) + b5_ref[t]

        outT = out[:, :_DG + 20].T
        noise_hatT = jnp.clip(outT[:_DG, :], -5.0, 5.0)
        alpha = _ALPHAS[t]
        beta = _BETAS[t]
        ab_t = _AB[t]
        ab_tm1 = _AB[t - 1]
        muT = (xgT - beta / jnp.sqrt(1.0 - ab_t) * noise_hatT) / jnp.sqrt(alpha)
        sigma = jnp.maximum(jnp.sqrt(beta * (1.0 - ab_tm1) / (1.0 - ab_t)), eps)
        s = _T + (_T - 1 - t)
        noiseT = normalsT[16 * s:16 * s + _DG, :]
        xgT = muT + sigma * noiseT

        for i, k in enumerate(_CATK):
            o0 = _DG + _GOFF[i]
            seg = outT[o0:o0 + k, :]
            m = jnp.max(seg, axis=0, keepdims=True)
            unn = jnp.exp(seg - m)
            cat_hat = unn / unn.sum(axis=0, keepdims=True)
            pi = (alpha * cats[i] + (1.0 - alpha) / k) * (ab_tm1 * cat_hat + (1.0 - ab_tm1) / k)
            pi = pi / (pi.sum(axis=0, keepdims=True) + eps)
            pi = jnp.maximum(pi, 1e-6)
            g0 = 20 * s + _GOFF[i]
            cats[i] = _cat_sample_t(jnp.log(pi), gumbelsT[g0:g0 + k, :], k)

    out_ref[...] = jnp.concatenate([xgT] + cats, axis=0).T


def kernel(x, W1, b1, g1, be1, W2, b2, g2, be2, W5, b5):
    full = lambda shape: pl.BlockSpec(shape, lambda i: (0,) * len(shape))
    return pl.pallas_call(
        _body,
        grid=(_GRID,),
        in_specs=[
            pl.BlockSpec((_RBLK, _DXIN), lambda i: (i, 0)),
            full((_T, _DIN, _H)), full((_T, _H)), full((_T, _H)), full((_T, _H)),
            full((_T, _H, _H)), full((_T, _H)), full((_T, _H)), full((_T, _H)),
            full((_T, _H, _DOUT + 1)), full((_T, _DOUT + 1)),
            full((_NW, 4)), full((_GW, 4)),
        ],
        out_specs=pl.BlockSpec((_RBLK, _DOUT), lambda i: (i, 0)),
        out_shape=jax.ShapeDtypeStruct((_B, _DOUT), jnp.float32),
    )(x, W1, b1, g1, be1, W2, b2, g2, be2, W5, b5,
      jnp.asarray(_RNGN), jnp.asarray(_RNGT))
